# baseline (speedup 1.0000x reference)
import functools

import jax
import jax.numpy as jnp
from jax import lax
from jax.experimental import pallas as pl
from jax.experimental.pallas import tpu as pltpu
from jax.experimental.pallas import tpu_sc as plsc

N_NODES = 10000
N_EDGES = 160000
D = 256
F = 128
NC = 2
NS = 16
L = 16
E_PAD = 163840
EPW = E_PAD // NS
K = 32
NCHUNK = EPW // K
NB = 4
IB = NCHUNK // NB
EQ = IB * K
RQ = EQ // F
A_ROWS = N_NODES + NS
ROWS_PER_CP = 80
NROW_CHUNKS = N_NODES // ROWS_PER_CP
NCP_ITERS = -(-NROW_CHUNKS // NS)
NGRP = K // L

RB = 1000
GRID = N_NODES // RB


def _stage1_body(nf_ref, we0_ref, we1_ref, be_ref, wn0_ref, bn_ref,
                 a2_ref, b2_ref, p_ref):
    x = nf_ref[...]
    a = jnp.dot(x, we0_ref[...], preferred_element_type=jnp.float32) + be_ref[...]
    b = jnp.dot(x, we1_ref[...], preferred_element_type=jnp.float32)
    p = jnp.dot(x, wn0_ref[...], preferred_element_type=jnp.float32) + bn_ref[...]
    a2_ref[0] = a[:, :F]
    a2_ref[1] = a[:, F:]
    b2_ref[0] = b[:, :F]
    b2_ref[1] = b[:, F:]
    p_ref[...] = p


def _stage1(nf, we0, we1, be, wn0, bn):
    return pl.pallas_call(
        _stage1_body,
        grid=(GRID,),
        in_specs=[
            pl.BlockSpec((RB, D), lambda i: (i, 0)),
            pl.BlockSpec((D, D), lambda i: (0, 0)),
            pl.BlockSpec((D, D), lambda i: (0, 0)),
            pl.BlockSpec((1, D), lambda i: (0, 0)),
            pl.BlockSpec((D, D), lambda i: (0, 0)),
            pl.BlockSpec((1, D), lambda i: (0, 0)),
        ],
        out_specs=[
            pl.BlockSpec((NC, RB, F), lambda i: (0, i, 0)),
            pl.BlockSpec((NC, RB, F), lambda i: (0, i, 0)),
            pl.BlockSpec((RB, D), lambda i: (i, 0)),
        ],
        out_shape=[
            jax.ShapeDtypeStruct((NC, N_NODES, F), jnp.float32),
            jax.ShapeDtypeStruct((NC, N_NODES, F), jnp.float32),
            jax.ShapeDtypeStruct((N_NODES, D), jnp.float32),
        ],
    )(nf, we0, we1, be, wn0, bn)


def _edge_body(a2, b2, wt2, src4, dst4, traj4,
               red2,
               src_v, dst_v, traj_v,
               ab0, ab1, m0, m1, wt_v,
               acc,
               ga0, ga1, gb0, gb1, sc0, sc1):
    c = lax.axis_index("c")
    s = lax.axis_index("s")
    ab_bufs, m_bufs = (ab0, ab1), (m0, m1)
    gsem_a, gsem_b, ssem = (ga0, ga1), (gb0, gb1), (sc0, sc1)

    pltpu.sync_copy(wt2.at[c], wt_v)

    zero = jnp.zeros((L,), jnp.float32)

    def _zero_row(e, _):
        for j in range(F // L):
            m0[e, pl.ds(j * L, L)] = zero
        return 0

    lax.fori_loop(0, K, _zero_row, 0)

    n_zch = A_ROWS // K

    def _zero_acc(k, _):
        i = s + k * NS

        @pl.when(i < n_zch)
        def _():
            pltpu.sync_copy(m0, acc.at[pl.ds(i * K, K)])

        return 0

    lax.fori_loop(0, -(-n_zch // NS), _zero_acc, 0)
    plsc.subcore_barrier()

    wt = [wt_v[pl.ds(j * L, L)] for j in range(F // L)]

    def _issue_b(i, p):
        pltpu.async_copy(b2.at[c].at[dst_v.at[i]], ab_bufs[p], gsem_b[p])

    def _issue_a_add(i, p):
        sidx = src_v.at[i // NB, pl.ds((i % NB) * K, K)]
        pltpu.async_copy(a2.at[c].at[sidx], ab_bufs[p], gsem_a[p], add=True)

    def _wait_b(p):
        pltpu.make_async_copy(a2.at[c].at[pl.ds(0, K)], ab_bufs[p], gsem_b[p]).wait()

    def _wait_a(p):
        pltpu.make_async_copy(a2.at[c].at[pl.ds(0, K)], ab_bufs[p], gsem_a[p]).wait()

    def _wait_scatter(p):
        pltpu.make_async_copy(a2.at[c].at[pl.ds(0, K)], m_bufs[p], ssem[p]).wait()

    def _compute_chunk(i, p):
        ab_v, m_v = ab_bufs[p], m_bufs[p]

        def _grp(g, _):
            tvec = traj_v[i // NB, pl.ds((i % NB) * K + g * L, L)]
            for e16 in range(L):
                e = g * L + e16
                t = tvec[e16]
                for j in range(F // L):
                    sl = pl.ds(j * L, L)
                    x = ab_v[e, sl] + t * wt[j]
                    m_v[e, sl] = jnp.maximum(x, x * 0.01)
            return 0

        lax.fori_loop(0, NGRP, _grp, 0)

    def _quarter(h, _):
        pltpu.sync_copy(src4.at[s * NB + h], src_v)
        pltpu.sync_copy(dst4.at[s * NB + h], dst_v)
        pltpu.sync_copy(traj4.at[s * NB + h], traj_v)
        _issue_b(0, 0)
        _issue_b(1, 1)
        _wait_b(0)
        _issue_a_add(0, 0)

        def _pair(k, _):
            for p in range(2):
                i = 2 * k + p
                _wait_a(p)

                @pl.when(i >= 2)
                def _():
                    _wait_scatter(p)

                _compute_chunk(i, p)
                pltpu.async_copy(m_bufs[p], acc.at[dst_v.at[i]], ssem[p],
                                 add=True)

                @pl.when(i + 2 < IB)
                def _():
                    _issue_b(i + 2, p)

                @pl.when(i + 1 < IB)
                def _():
                    _wait_b(1 - p)
                    _issue_a_add(i + 1, 1 - p)

            return 0

        lax.fori_loop(0, IB // 2, _pair, 0)
        _wait_scatter(0)
        _wait_scatter(1)
        return 0

    lax.fori_loop(0, NB, _quarter, 0)
    plsc.subcore_barrier()

    def _out(k, _):
        i = s + k * NS

        @pl.when(i < NROW_CHUNKS)
        def _():
            r0 = i * ROWS_PER_CP
            pltpu.sync_copy(acc.at[pl.ds(r0, ROWS_PER_CP)],
                            red2.at[c].at[pl.ds(r0, ROWS_PER_CP)])

        return 0

    lax.fori_loop(0, NCP_ITERS, _out, 0)


def _stage2(a2, b2, wt2, src4, dst4, traj4):
    mesh = plsc.VectorSubcoreMesh(core_axis_name="c", subcore_axis_name="s")
    f = pl.kernel(
        _edge_body,
        out_type=jax.ShapeDtypeStruct((NC, N_NODES, F), jnp.float32),
        mesh=mesh,
        scratch_types=[
            pltpu.VMEM((RQ, F), jnp.int32),
            pltpu.VMEM((IB, K), jnp.int32),
            pltpu.VMEM((RQ, F), jnp.float32),
            pltpu.VMEM((K, F), jnp.float32),
            pltpu.VMEM((K, F), jnp.float32),
            pltpu.VMEM((K, F), jnp.float32),
            pltpu.VMEM((K, F), jnp.float32),
            pltpu.VMEM((F,), jnp.float32),
            pltpu.VMEM_SHARED((A_ROWS, F), jnp.float32),
            pltpu.SemaphoreType.DMA,
            pltpu.SemaphoreType.DMA,
            pltpu.SemaphoreType.DMA,
            pltpu.SemaphoreType.DMA,
            pltpu.SemaphoreType.DMA,
            pltpu.SemaphoreType.DMA,
        ],
    )
    return f(a2, b2, wt2, src4, dst4, traj4)


def _stage3_body(p_ref, red2_ref, wn1_ref, out_ref):
    r0 = red2_ref[0]
    r1 = red2_ref[1]
    y = (p_ref[...]
         + jnp.dot(r0, wn1_ref[0], preferred_element_type=jnp.float32)
         + jnp.dot(r1, wn1_ref[1], preferred_element_type=jnp.float32))
    out_ref[...] = jnp.where(y > 0.0, y, y * 0.01)


def _stage3(p, red2, wn1):
    return pl.pallas_call(
        _stage3_body,
        grid=(GRID,),
        in_specs=[
            pl.BlockSpec((RB, D), lambda i: (i, 0)),
            pl.BlockSpec((NC, RB, F), lambda i: (0, i, 0)),
            pl.BlockSpec((NC, F, D), lambda i: (0, 0, 0)),
        ],
        out_specs=pl.BlockSpec((RB, D), lambda i: (i, 0)),
        out_shape=jax.ShapeDtypeStruct((N_NODES, D), jnp.float32),
    )(p, red2, wn1)


@jax.jit
def kernel(nf, edge_index, traj, W_e, b_e, W_n, b_n):
    src = edge_index[0].astype(jnp.int32)
    dst = edge_index[1].astype(jnp.int32)
    we0 = W_e[:D]
    we1 = W_e[D:2 * D]
    wt2 = W_e[2 * D].reshape(NC, F)
    wn0 = W_n[:D]
    wn1 = W_n[D:].reshape(NC, F, D)
    pad = E_PAD - N_EDGES
    src = jnp.concatenate([src, jnp.zeros((pad,), jnp.int32)])
    dst = jnp.concatenate(
        [dst, N_NODES + (jnp.arange(pad, dtype=jnp.int32) % NS)])
    traj = jnp.concatenate([traj, jnp.zeros((pad,), jnp.float32)])
    src4 = src.reshape(NS * NB, RQ, F)
    dst4 = dst.reshape(NS * NB, IB, K)
    traj4 = traj.reshape(NS * NB, RQ, F)
    a2, b2, p = _stage1(nf, we0, we1, b_e.reshape(1, D), wn0, b_n.reshape(1, D))
    red2 = _stage2(a2, b2, wt2, src4, dst4, traj4)
    return _stage3(p, red2, wn1)


# speedup vs baseline: 1.8171x; 1.8171x over previous
import functools

import jax
import jax.numpy as jnp
from jax import lax
from jax.experimental import pallas as pl
from jax.experimental.pallas import tpu as pltpu
from jax.experimental.pallas import tpu_sc as plsc

N_NODES = 10000
N_EDGES = 160000
D = 256
F = 128
NC = 2
NS = 16
L = 16
E_PAD = 163840
EPW = E_PAD // NS
K = 32
K2 = 2 * K
NCHUNK = EPW // K
NB = 4
IB = NCHUNK // NB
EQ = IB * K
SDR = IB * K2 // F
TR = EQ // F
A_ROWS = N_NODES + NS
ROWS_PER_CP = 80
NROW_CHUNKS = N_NODES // ROWS_PER_CP
NCP_ITERS = -(-NROW_CHUNKS // NS)
NGRP = K // L

RB = 1000
GRID = -(-A_ROWS // RB)


def _stage1_body(nf_ref, we0_ref, we1_ref, be_ref, wn0_ref, bn_ref,
                 t_ref, p_ref):
    x = nf_ref[...]
    a = jnp.dot(x, we0_ref[...], preferred_element_type=jnp.float32) + be_ref[...]
    b = jnp.dot(x, we1_ref[...], preferred_element_type=jnp.float32)
    p = jnp.dot(x, wn0_ref[...], preferred_element_type=jnp.float32) + bn_ref[...]
    t_ref[0, 0] = a[:, :F]
    t_ref[1, 0] = a[:, F:]
    t_ref[0, 1] = b[:, :F]
    t_ref[1, 1] = b[:, F:]
    p_ref[...] = p


def _stage1(nf, we0, we1, be, wn0, bn):
    return pl.pallas_call(
        _stage1_body,
        grid=(GRID,),
        in_specs=[
            pl.BlockSpec((RB, D), lambda i: (i, 0)),
            pl.BlockSpec((D, D), lambda i: (0, 0)),
            pl.BlockSpec((D, D), lambda i: (0, 0)),
            pl.BlockSpec((1, D), lambda i: (0, 0)),
            pl.BlockSpec((D, D), lambda i: (0, 0)),
            pl.BlockSpec((1, D), lambda i: (0, 0)),
        ],
        out_specs=[
            pl.BlockSpec((NC, 2, RB, F), lambda i: (0, 0, i, 0)),
            pl.BlockSpec((RB, D), lambda i: (i, 0)),
        ],
        out_shape=[
            jax.ShapeDtypeStruct((NC, 2, A_ROWS, F), jnp.float32),
            jax.ShapeDtypeStruct((N_NODES, D), jnp.float32),
        ],
    )(nf, we0, we1, be, wn0, bn)


def _edge_body(t2, wt2, sd4, dst4, traj4,
               red2,
               sd_v, dst_v, traj_v,
               ab0, ab1, m0, m1, wt_v,
               acc,
               ga0, ga1, sc0, sc1):
    c = lax.axis_index("c")
    s = lax.axis_index("s")
    ab_bufs, m_bufs = (ab0, ab1), (m0, m1)
    gsem, ssem = (ga0, ga1), (sc0, sc1)

    pltpu.sync_copy(wt2.at[c], wt_v)

    zero = jnp.zeros((L,), jnp.float32)

    def _zero_row(e, _):
        for j in range(F // L):
            m0[e, pl.ds(j * L, L)] = zero
        return 0

    lax.fori_loop(0, K, _zero_row, 0)

    n_zch = A_ROWS // K

    def _zero_acc(k, _):
        i = s + k * NS

        @pl.when(i < n_zch)
        def _():
            pltpu.sync_copy(m0, acc.at[pl.ds(i * K, K)])

        return 0

    lax.fori_loop(0, -(-n_zch // NS), _zero_acc, 0)
    plsc.subcore_barrier()

    wt = [wt_v[pl.ds(j * L, L)] for j in range(F // L)]

    def _issue_gather(i, p):
        sidx = sd_v.at[i // 2, pl.ds((i % 2) * K2, K2)]
        pltpu.async_copy(t2.at[c].at[sidx], ab_bufs[p], gsem[p])

    def _wait_gather(p):
        pltpu.make_async_copy(t2.at[c].at[pl.ds(0, K2)], ab_bufs[p],
                              gsem[p]).wait()

    def _wait_scatter(p):
        pltpu.make_async_copy(red2.at[c].at[pl.ds(0, K)], m_bufs[p],
                              ssem[p]).wait()

    def _compute_chunk(i, p):
        ab_v, m_v = ab_bufs[p], m_bufs[p]

        def _grp(g, _):
            tflat = i * K + g * L
            tvec = traj_v[tflat // F, pl.ds(tflat % F, L)]
            for e16 in range(L):
                e = g * L + e16
                t = tvec[e16]
                for j in range(F // L):
                    sl = pl.ds(j * L, L)
                    x = ab_v[e, sl] + ab_v[K + e, sl] + t * wt[j]
                    m_v[e, sl] = jnp.maximum(x, x * 0.01)
            return 0

        lax.fori_loop(0, NGRP, _grp, 0)

    def _quarter(h, _):
        pltpu.sync_copy(sd4.at[s * NB + h], sd_v)
        pltpu.sync_copy(dst4.at[s * NB + h], dst_v)
        pltpu.sync_copy(traj4.at[s * NB + h], traj_v)
        _issue_gather(0, 0)
        _issue_gather(1, 1)

        def _pair(k, _):
            for p in range(2):
                i = 2 * k + p
                _wait_gather(p)

                @pl.when(i >= 2)
                def _():
                    _wait_scatter(p)

                _compute_chunk(i, p)
                pltpu.async_copy(m_bufs[p], acc.at[dst_v.at[i]], ssem[p],
                                 add=True)

                @pl.when(i + 2 < IB)
                def _():
                    _issue_gather(i + 2, p)

            return 0

        lax.fori_loop(0, IB // 2, _pair, 0)
        _wait_scatter(0)
        _wait_scatter(1)
        return 0

    lax.fori_loop(0, NB, _quarter, 0)
    plsc.subcore_barrier()

    def _out(k, _):
        i = s + k * NS

        @pl.when(i < NROW_CHUNKS)
        def _():
            r0 = i * ROWS_PER_CP
            pltpu.sync_copy(acc.at[pl.ds(r0, ROWS_PER_CP)],
                            red2.at[c].at[pl.ds(r0, ROWS_PER_CP)])

        return 0

    lax.fori_loop(0, NCP_ITERS, _out, 0)


def _stage2(t2, wt2, sd4, dst4, traj4):
    mesh = plsc.VectorSubcoreMesh(core_axis_name="c", subcore_axis_name="s")
    f = pl.kernel(
        _edge_body,
        out_type=jax.ShapeDtypeStruct((NC, N_NODES, F), jnp.float32),
        mesh=mesh,
        scratch_types=[
            pltpu.VMEM((SDR, F), jnp.int32),
            pltpu.VMEM((IB, K), jnp.int32),
            pltpu.VMEM((TR, F), jnp.float32),
            pltpu.VMEM((K2, F), jnp.float32),
            pltpu.VMEM((K2, F), jnp.float32),
            pltpu.VMEM((K, F), jnp.float32),
            pltpu.VMEM((K, F), jnp.float32),
            pltpu.VMEM((F,), jnp.float32),
            pltpu.VMEM_SHARED((A_ROWS, F), jnp.float32),
            pltpu.SemaphoreType.DMA,
            pltpu.SemaphoreType.DMA,
            pltpu.SemaphoreType.DMA,
            pltpu.SemaphoreType.DMA,
        ],
    )
    return f(t2, wt2, sd4, dst4, traj4)


def _stage3_body(p_ref, red2_ref, wn1_ref, out_ref):
    r0 = red2_ref[0]
    r1 = red2_ref[1]
    y = (p_ref[...]
         + jnp.dot(r0, wn1_ref[0], preferred_element_type=jnp.float32)
         + jnp.dot(r1, wn1_ref[1], preferred_element_type=jnp.float32))
    out_ref[...] = jnp.where(y > 0.0, y, y * 0.01)


def _stage3(p, red2, wn1):
    return pl.pallas_call(
        _stage3_body,
        grid=(N_NODES // RB,),
        in_specs=[
            pl.BlockSpec((RB, D), lambda i: (i, 0)),
            pl.BlockSpec((NC, RB, F), lambda i: (0, i, 0)),
            pl.BlockSpec((NC, F, D), lambda i: (0, 0, 0)),
        ],
        out_specs=pl.BlockSpec((RB, D), lambda i: (i, 0)),
        out_shape=jax.ShapeDtypeStruct((N_NODES, D), jnp.float32),
    )(p, red2, wn1)


@jax.jit
def kernel(nf, edge_index, traj, W_e, b_e, W_n, b_n):
    src = edge_index[0].astype(jnp.int32)
    dst = edge_index[1].astype(jnp.int32)
    we0 = W_e[:D]
    we1 = W_e[D:2 * D]
    wt2 = W_e[2 * D].reshape(NC, F)
    wn0 = W_n[:D]
    wn1 = W_n[D:].reshape(NC, F, D)
    pad = E_PAD - N_EDGES
    pad_rows = N_NODES + (jnp.arange(pad, dtype=jnp.int32) % NS)
    src = jnp.concatenate([src, pad_rows])
    dst = jnp.concatenate([dst, pad_rows])
    traj = jnp.concatenate([traj, jnp.zeros((pad,), jnp.float32)])
    sd = jnp.concatenate(
        [src.reshape(-1, K), dst.reshape(-1, K) + A_ROWS], axis=1)
    sd4 = sd.reshape(NS * NB, SDR, F)
    dst4 = dst.reshape(NS * NB, IB, K)
    traj4 = traj.reshape(NS * NB, TR, F)
    t, p = _stage1(nf, we0, we1, b_e.reshape(1, D), wn0, b_n.reshape(1, D))
    t2 = t.reshape(NC, 2 * A_ROWS, F)
    red2 = _stage2(t2, wt2, sd4, dst4, traj4)
    return _stage3(p, red2, wn1)
